# baseline (speedup 1.0000x reference)
import functools

import jax
import jax.numpy as jnp
from jax import lax
from jax.experimental import pallas as pl
from jax.experimental.pallas import tpu as pltpu
from jax.experimental.pallas import tpu_sc as plsc

F32 = jnp.float32
I32 = jnp.int32
NC = 2
NS = 16
NW = NC * NS
EK = 128
D = 16
BT = 2000


_SC_PARAMS = pltpu.CompilerParams(
    needs_layout_passes=False, use_tc_tiling_on_sc=False)


def _sc_mesh():
  return plsc.VectorSubcoreMesh(
      core_axis_name="c", subcore_axis_name="s", num_cores=NC, num_subcores=NS)


def _worker_span(w, nrows_total):
  q, rem = divmod(nrows_total, NW)
  n = q + jnp.where(w < rem, 1, 0)
  base = q * w + jnp.minimum(w, rem)
  return base, n


def _deg_call(dst_rows, n_nodes):
  nrows_total = dst_rows.shape[0]
  nchunk = n_nodes // 2000

  NB = 5

  @functools.partial(
      pl.kernel,
      out_type=[jax.ShapeDtypeStruct((n_nodes,), F32)] * NC,
      mesh=_sc_mesh(),
      compiler_params=_SC_PARAMS,
      scratch_types=[pltpu.VMEM((1, EK), I32)] * NB
      + [
          pltpu.VMEM((EK,), F32),
          pltpu.VMEM((2000,), F32),
          pltpu.VMEM_SHARED((n_nodes,), F32),
      ]
      + [pltpu.SemaphoreType.DMA] * (2 * NB),
  )
  def deg_kernel(dst_hbm, deg0_hbm, deg1_hbm, db0, db1, db2, db3, db4,
                 ones_v, zbuf, acc, el0, el1, el2, el3, el4,
                 sl0, sl1, sl2, sl3, sl4):
    dbufs = (db0, db1, db2, db3, db4)
    esem = (el0, el1, el2, el3, el4)
    ssem = (sl0, sl1, sl2, sl3, sl4)
    cid = lax.axis_index("c")
    sid = lax.axis_index("s")
    w = cid * NS + sid
    for i in range(0, EK, 16):
      ones_v[pl.ds(i, 16)] = jnp.ones((16,), F32)
    for i in range(0, 2000, 16):
      zbuf[pl.ds(i, 16)] = jnp.zeros((16,), F32)

    @pl.loop(sid, nchunk, step=NS)
    def _(j):
      pltpu.sync_copy(zbuf, acc.at[pl.ds(j * 2000, 2000)])

    plsc.subcore_barrier()
    base, n = _worker_span(w, nrows_total)

    for k in range(3):
      pltpu.async_copy(dst_hbm.at[base + k], dbufs[k], esem[k])

    @pl.loop(0, (n + NB - 1) // NB)
    def _(i):
      for b0 in range(NB):
        s = i * NB + b0
        b3 = (b0 + 3) % NB

        @pl.when(jnp.logical_and(s >= 2, s < n))
        def _():
          pltpu.make_async_copy(
              ones_v, acc.at[dbufs[b3].at[0]], ssem[b3]).wait()

        @pl.when(s + 3 < n)
        def _():
          pltpu.async_copy(dst_hbm.at[base + s + 3], dbufs[b3], esem[b3])

        @pl.when(s < n)
        def _():
          pltpu.make_async_copy(dst_hbm.at[base + s], dbufs[b0],
                                esem[b0]).wait()
          pltpu.async_copy(ones_v, acc.at[dbufs[b0].at[0]], ssem[b0],
                           add=True)

    for b in range(NB):
      for tail in (n - 2, n - 1):
        @pl.when(tail % NB == b)
        def _():
          pltpu.make_async_copy(ones_v, acc.at[dbufs[b].at[0]],
                                ssem[b]).wait()

    plsc.subcore_barrier()
    for cc, deg_hbm in enumerate((deg0_hbm, deg1_hbm)):
      @pl.when(cid == cc)
      def _():
        @pl.loop(sid, nchunk, step=NS)
        def _(j):
          pltpu.sync_copy(acc.at[pl.ds(j * 2000, 2000)], zbuf)
          pltpu.sync_copy(zbuf, deg_hbm.at[pl.ds(j * 2000, 2000)])

  return deg_kernel(dst_rows)


def _msg_call(hs, edata, n_nodes):
  nrows_total = edata.shape[0]
  rps = n_nodes // NS
  zrows = 250
  nzc = n_nodes // zrows

  one_f32_bits = 0x3F800000

  NB = 5

  @functools.partial(
      pl.kernel,
      out_type=[jax.ShapeDtypeStruct((n_nodes, D), F32)] * NC,
      mesh=_sc_mesh(),
      compiler_params=_SC_PARAMS,
      scratch_types=[pltpu.VMEM((3, EK), I32)] * NB
      + [pltpu.VMEM((EK, D), F32)] * NB
      + [
          pltpu.VMEM((zrows, D), F32),
          pltpu.VMEM_SHARED((n_nodes, D), F32),
      ]
      + [pltpu.SemaphoreType.DMA] * (3 * NB),
  )
  def msg_kernel(hs_hbm, edata_hbm, out0_hbm, out1_hbm, eb0, eb1, eb2, eb3,
                 eb4, rw0, rw1, rw2, rw3, rw4, zbuf, acc,
                 el0, el1, el2, el3, el4, gl0, gl1, gl2, gl3, gl4,
                 sl0, sl1, sl2, sl3, sl4):
    ebufs = (eb0, eb1, eb2, eb3, eb4)
    rows = (rw0, rw1, rw2, rw3, rw4)
    esem = (el0, el1, el2, el3, el4)
    gsem = (gl0, gl1, gl2, gl3, gl4)
    ssem = (sl0, sl1, sl2, sl3, sl4)
    cid = lax.axis_index("c")
    sid = lax.axis_index("s")
    w = cid * NS + sid
    for i in range(zrows):
      zbuf[i] = jnp.zeros((16,), F32)

    @pl.loop(sid, nzc, step=NS)
    def _(j):
      pltpu.sync_copy(zbuf, acc.at[pl.ds(j * zrows, zrows)])

    plsc.subcore_barrier()
    base, n = _worker_span(w, nrows_total)

    for k in range(3):
      pltpu.async_copy(edata_hbm.at[base + k], ebufs[k], esem[k])
    for k in range(2):
      pltpu.make_async_copy(edata_hbm.at[base + k], ebufs[k], esem[k]).wait()
      pltpu.async_copy(hs_hbm.at[ebufs[k].at[0]], rows[k], gsem[k])

    @pl.loop(0, (n + NB - 1) // NB)
    def _(i):
      for b0 in range(NB):
        s = i * NB + b0
        b2 = (b0 + 2) % NB
        b3 = (b0 + 3) % NB

        @pl.when(s < n)
        def _():
          pltpu.make_async_copy(hs_hbm.at[ebufs[b0].at[0]], rows[b0],
                                gsem[b0]).wait()

        @pl.when(jnp.logical_and(s >= 2, s < n))
        def _():
          pltpu.make_async_copy(rows[b3], acc.at[ebufs[b3].at[1]],
                                ssem[b3]).wait()

        @pl.when(s + 3 < n)
        def _():
          pltpu.async_copy(edata_hbm.at[base + s + 3], ebufs[b3], esem[b3])

        @pl.when(s + 2 < n)
        def _():
          pltpu.make_async_copy(edata_hbm.at[base + s + 2], ebufs[b2],
                                esem[b2]).wait()
          pltpu.async_copy(hs_hbm.at[ebufs[b2].at[0]], rows[b2], gsem[b2])

        @pl.when(s < n)
        def _():
          ones_chunk = jnp.full((16,), one_f32_bits, I32)
          all_ones = jnp.all(ebufs[b0][2, pl.ds(0, 16)] == ones_chunk)
          for gch in range(1, EK // 16):
            all_ones = jnp.logical_and(
                all_ones,
                jnp.all(ebufs[b0][2, pl.ds(gch * 16, 16)] == ones_chunk))

          @pl.when(jnp.logical_not(all_ones))
          def _():
            for c in range(EK):
              mb = plsc.load_gather(
                  ebufs[b0],
                  [jnp.full((16,), 2, I32), jnp.full((16,), c, I32)])
              rows[b0][c] = rows[b0][c] * plsc.bitcast(mb, F32)

          pltpu.async_copy(rows[b0], acc.at[ebufs[b0].at[1]], ssem[b0],
                           add=True)

    for b in range(NB):
      for tail in (n - 2, n - 1):
        @pl.when(tail % NB == b)
        def _():
          pltpu.make_async_copy(rows[b], acc.at[ebufs[b].at[1]],
                                ssem[b]).wait()

    plsc.subcore_barrier()
    for cc, out_hbm in enumerate((out0_hbm, out1_hbm)):
      @pl.when(cid == cc)
      def _():
        @pl.loop(0, rps // zrows)
        def _(j):
          off = sid * rps + j * zrows
          pltpu.sync_copy(acc.at[pl.ds(off, zrows)], zbuf)
          pltpu.sync_copy(zbuf, out_hbm.at[pl.ds(off, zrows)])

  return msg_kernel(hs, edata)


def _tc_h(x1, x2, rd, W1):
  n = x1.shape[0]
  g = n // BT

  def body(x1_r, x2_r, rd_r, w_r, h_r):
    w = w_r[...]
    h_r[...] = (
        jnp.dot(x1_r[...], w[0:16], preferred_element_type=F32)
        + jnp.dot(x2_r[...], w[16:32], preferred_element_type=F32)
        + jnp.dot(rd_r[...], w[32:48], preferred_element_type=F32))

  return pl.pallas_call(
      body,
      grid=(g,),
      in_specs=[pl.BlockSpec((BT, 16), lambda i: (i, 0))] * 3
      + [pl.BlockSpec((48, 16), lambda i: (0, 0))],
      out_specs=pl.BlockSpec((BT, D), lambda i: (i, 0)),
      out_shape=jax.ShapeDtypeStruct((n, D), F32),
  )(x1, x2, rd, W1)


def _dinv(d):
  return jnp.where(d > 0, 1.0 / jnp.sqrt(jnp.maximum(d, 1e-12)), 0.0)


def _tc_hs(deg0, deg1, h):
  n = h.shape[0]
  g = n // BT

  def body(d0_r, d1_r, h_r, hs_r):
    dinv = _dinv(d0_r[0, 0, :] + d1_r[0, 0, :])
    hs_r[...] = h_r[...] * dinv[:, None]

  return pl.pallas_call(
      body,
      grid=(g,),
      in_specs=[pl.BlockSpec((1, 1, BT), lambda i: (i, 0, 0))] * 2
      + [pl.BlockSpec((BT, D), lambda i: (i, 0))],
      out_specs=pl.BlockSpec((BT, D), lambda i: (i, 0)),
      out_shape=jax.ShapeDtypeStruct((n, D), F32),
  )(deg0, deg1, h)


def _elu(x):
  return jnp.where(x > 0, x, jnp.exp(jnp.minimum(x, 0.0)) - 1.0)


def _stats_update(st_r, y):
  s = jnp.sum(y, axis=0)
  ss = jnp.sum(y * y, axis=0)
  upd = jnp.concatenate(
      [s[None, :], ss[None, :], jnp.zeros((6, y.shape[1]), F32)], axis=0)
  st_r[...] += upd


def _fold(st_r, g, b, nf):
  mu = st_r[0, :] / nf
  var = jnp.maximum(st_r[1, :] / nf - mu * mu, 0.0)
  si = g / jnp.sqrt(var + 1e-5)
  return si, b - mu * si


def _tc_conv_finish(deg0, deg1, p0, p1, x1, x2, b1):
  n = x1.shape[0]
  g = n // BT

  def body(d0_r, d1_r, p0_r, p1_r, x1_r, x2_r, b1_r, he_r, st_r):
    @pl.when(pl.program_id(0) == 0)
    def _():
      st_r[...] = jnp.zeros_like(st_r)

    dinv = _dinv(d0_r[0, 0, :] + d1_r[0, 0, :])
    conv = (p0_r[...] + p1_r[...]) * dinv[:, None] + b1_r[...]
    he = _elu(conv)
    he_r[...] = he
    y = jnp.concatenate([x1_r[...], x2_r[...], he], axis=1)
    _stats_update(st_r, y)

  return pl.pallas_call(
      body,
      grid=(g,),
      in_specs=[pl.BlockSpec((1, 1, BT), lambda i: (i, 0, 0))] * 2
      + [pl.BlockSpec((BT, D), lambda i: (i, 0))] * 4
      + [pl.BlockSpec((1, D), lambda i: (0, 0))],
      out_specs=(pl.BlockSpec((BT, D), lambda i: (i, 0)),
                 pl.BlockSpec((8, 48), lambda i: (0, 0))),
      out_shape=(jax.ShapeDtypeStruct((n, D), F32),
                 jax.ShapeDtypeStruct((8, 48), F32)),
  )(deg0, deg1, p0, p1, x1, x2, b1)


def _tc_mlp0(x1, x2, he, A0, c0):
  n = x1.shape[0]
  g = n // BT

  def body(x1_r, x2_r, he_r, a_r, c_r, a0_r, st_r):
    @pl.when(pl.program_id(0) == 0)
    def _():
      st_r[...] = jnp.zeros_like(st_r)

    y = jnp.concatenate([x1_r[...], x2_r[...], he_r[...]], axis=1)
    a0 = _elu(jnp.dot(y, a_r[...], preferred_element_type=F32) + c_r[...])
    a0_r[...] = a0
    _stats_update(st_r, a0)

  return pl.pallas_call(
      body,
      grid=(g,),
      in_specs=[pl.BlockSpec((BT, D), lambda i: (i, 0))] * 3
      + [pl.BlockSpec((48, 16), lambda i: (0, 0)),
         pl.BlockSpec((1, 16), lambda i: (0, 0))],
      out_specs=(pl.BlockSpec((BT, 16), lambda i: (i, 0)),
                 pl.BlockSpec((8, 16), lambda i: (0, 0))),
      out_shape=(jax.ShapeDtypeStruct((n, 16), F32),
                 jax.ShapeDtypeStruct((8, 16), F32)),
  )(x1, x2, he, A0, c0)


def _tc_mlp1(a0, A1, c1):
  n = a0.shape[0]
  g = n // BT

  def body(a0_r, a_r, c_r, a1_r, st_r):
    @pl.when(pl.program_id(0) == 0)
    def _():
      st_r[...] = jnp.zeros_like(st_r)

    a1 = _elu(jnp.dot(a0_r[...], a_r[...], preferred_element_type=F32)
              + c_r[...])
    a1_r[...] = a1
    _stats_update(st_r, a1)

  return pl.pallas_call(
      body,
      grid=(g,),
      in_specs=[pl.BlockSpec((BT, 16), lambda i: (i, 0)),
                pl.BlockSpec((16, 8), lambda i: (0, 0)),
                pl.BlockSpec((1, 8), lambda i: (0, 0))],
      out_specs=(pl.BlockSpec((BT, 8), lambda i: (i, 0)),
                 pl.BlockSpec((8, 8), lambda i: (0, 0))),
      out_shape=(jax.ShapeDtypeStruct((n, 8), F32),
                 jax.ShapeDtypeStruct((8, 8), F32)),
  )(a0, A1, c1)


def _tc_affine(a1, scale, off):
  n = a1.shape[0]
  g = n // BT

  def body(a1_r, s_r, o_r, out_r):
    out_r[...] = a1_r[...] * s_r[...] + o_r[...]

  return pl.pallas_call(
      body,
      grid=(g,),
      in_specs=[pl.BlockSpec((BT, 8), lambda i: (i, 0)),
                pl.BlockSpec((1, 8), lambda i: (0, 0)),
                pl.BlockSpec((1, 8), lambda i: (0, 0))],
      out_specs=pl.BlockSpec((BT, 8), lambda i: (i, 0)),
      out_shape=jax.ShapeDtypeStruct((n, 8), F32),
  )(a1, scale, off)


def kernel(x1, x2, batch, random_dims, x_j_mask, edge_index, W1, b1,
           bn1_g, bn1_b, mlp0_W, mlp0_b, bnm0_g, bnm0_b,
           mlp1_W, mlp1_b, bnm1_g, bnm1_b):
  del batch
  n = x1.shape[0]
  e = x_j_mask.shape[0]
  r = e // EK
  src = edge_index[0].astype(I32)
  dst = edge_index[1].astype(I32)
  dst_rows = dst.reshape(r, EK)
  mbits = lax.bitcast_convert_type(x_j_mask.astype(F32), I32)
  edata = jnp.stack([src.reshape(r, EK), dst_rows, mbits.reshape(r, EK)],
                    axis=1)

  deg0f, deg1f = _deg_call(dst_rows.reshape(r, 1, EK), n)
  h = _tc_h(x1, x2, random_dims, W1)
  d0 = deg0f.reshape(n // BT, 1, BT)
  d1 = deg1f.reshape(n // BT, 1, BT)
  hs = _tc_hs(d0, d1, h)
  p0, p1 = _msg_call(hs, edata, n)

  he, yst = _tc_conv_finish(d0, d1, p0, p1, x1, x2, b1.reshape(1, D))

  nf = float(n)
  mu0 = yst[0] / nf
  var0 = jnp.maximum(yst[1] / nf - mu0 * mu0, 0.0)
  s0inv = bn1_g / jnp.sqrt(var0 + 1e-5)
  A0 = s0inv[:, None] * mlp0_W
  c0 = (bn1_b - mu0 * s0inv) @ mlp0_W + mlp0_b

  a0, st0 = _tc_mlp0(x1, x2, he, A0, c0.reshape(1, 16))
  mu1 = st0[0] / nf
  var1 = jnp.maximum(st0[1] / nf - mu1 * mu1, 0.0)
  s1inv = bnm0_g / jnp.sqrt(var1 + 1e-5)
  A1 = s1inv[:, None] * mlp1_W
  c1 = (bnm0_b - mu1 * s1inv) @ mlp1_W + mlp1_b

  a1, st1 = _tc_mlp1(a0, A1, c1.reshape(1, 8))
  mu2 = st1[0] / nf
  var2 = jnp.maximum(st1[1] / nf - mu2 * mu2, 0.0)
  s2inv = bnm1_g / jnp.sqrt(var2 + 1e-5)
  off2 = bnm1_b - mu2 * s2inv

  return _tc_affine(a1, s2inv.reshape(1, 8), off2.reshape(1, 8))


# speedup vs baseline: 1.0615x; 1.0615x over previous
import functools

import jax
import jax.numpy as jnp
from jax import lax
from jax.experimental import pallas as pl
from jax.experimental.pallas import tpu as pltpu
from jax.experimental.pallas import tpu_sc as plsc

F32 = jnp.float32
I32 = jnp.int32
NC = 2
NS = 16
NW = NC * NS
EK = 128
D = 16
BT = 2000


_SC_PARAMS = pltpu.CompilerParams(
    needs_layout_passes=False, use_tc_tiling_on_sc=False)


def _sc_mesh():
  return plsc.VectorSubcoreMesh(
      core_axis_name="c", subcore_axis_name="s", num_cores=NC, num_subcores=NS)


def _worker_span(w, nrows_total):
  q, rem = divmod(nrows_total, NW)
  n = q + jnp.where(w < rem, 1, 0)
  base = q * w + jnp.minimum(w, rem)
  return base, n


def _deg_call(dst_rows, n_nodes):
  nrows_total = dst_rows.shape[0]
  nchunk = n_nodes // 2000

  NB = 5

  @functools.partial(
      pl.kernel,
      out_type=[jax.ShapeDtypeStruct((n_nodes,), F32)] * NC,
      mesh=_sc_mesh(),
      compiler_params=_SC_PARAMS,
      scratch_types=[pltpu.VMEM((1, EK), I32)] * NB
      + [
          pltpu.VMEM((EK,), F32),
          pltpu.VMEM((2000,), F32),
          pltpu.VMEM_SHARED((n_nodes,), F32),
      ]
      + [pltpu.SemaphoreType.DMA] * (2 * NB),
  )
  def deg_kernel(dst_hbm, deg0_hbm, deg1_hbm, db0, db1, db2, db3, db4,
                 ones_v, zbuf, acc, el0, el1, el2, el3, el4,
                 sl0, sl1, sl2, sl3, sl4):
    dbufs = (db0, db1, db2, db3, db4)
    esem = (el0, el1, el2, el3, el4)
    ssem = (sl0, sl1, sl2, sl3, sl4)
    cid = lax.axis_index("c")
    sid = lax.axis_index("s")
    w = cid * NS + sid
    for i in range(0, EK, 16):
      ones_v[pl.ds(i, 16)] = jnp.ones((16,), F32)
    for i in range(0, 2000, 16):
      zbuf[pl.ds(i, 16)] = jnp.zeros((16,), F32)

    @pl.loop(sid, nchunk, step=NS)
    def _(j):
      pltpu.sync_copy(zbuf, acc.at[pl.ds(j * 2000, 2000)])

    plsc.subcore_barrier()
    base, n = _worker_span(w, nrows_total)

    for k in range(3):
      pltpu.async_copy(dst_hbm.at[base + k], dbufs[k], esem[k])

    @pl.loop(0, (n + NB - 1) // NB)
    def _(i):
      for b0 in range(NB):
        s = i * NB + b0
        b3 = (b0 + 3) % NB

        @pl.when(jnp.logical_and(s >= 2, s < n))
        def _():
          pltpu.make_async_copy(
              ones_v, acc.at[dbufs[b3].at[0]], ssem[b3]).wait()

        @pl.when(s + 3 < n)
        def _():
          pltpu.async_copy(dst_hbm.at[base + s + 3], dbufs[b3], esem[b3])

        @pl.when(s < n)
        def _():
          pltpu.make_async_copy(dst_hbm.at[base + s], dbufs[b0],
                                esem[b0]).wait()
          pltpu.async_copy(ones_v, acc.at[dbufs[b0].at[0]], ssem[b0],
                           add=True)

    for b in range(NB):
      for tail in (n - 2, n - 1):
        @pl.when(tail % NB == b)
        def _():
          pltpu.make_async_copy(ones_v, acc.at[dbufs[b].at[0]],
                                ssem[b]).wait()

    plsc.subcore_barrier()
    for cc, deg_hbm in enumerate((deg0_hbm, deg1_hbm)):
      @pl.when(cid == cc)
      def _():
        @pl.loop(sid, nchunk, step=NS)
        def _(j):
          pltpu.sync_copy(acc.at[pl.ds(j * 2000, 2000)], zbuf)
          pltpu.sync_copy(zbuf, deg_hbm.at[pl.ds(j * 2000, 2000)])

  return deg_kernel(dst_rows)


def _msg_call(hs, edata, n_nodes):
  nrows_total = edata.shape[0]
  rps = n_nodes // NS
  zrows = 250
  nzc = n_nodes // zrows

  one_f32_bits = 0x3F800000

  NB = 5

  @functools.partial(
      pl.kernel,
      out_type=[jax.ShapeDtypeStruct((n_nodes, D), F32)] * NC,
      mesh=_sc_mesh(),
      compiler_params=_SC_PARAMS,
      scratch_types=[pltpu.VMEM((3, EK), I32)] * NB
      + [pltpu.VMEM((EK, D), F32)] * NB
      + [
          pltpu.VMEM((zrows, D), F32),
          pltpu.VMEM_SHARED((n_nodes, D), F32),
      ]
      + [pltpu.SemaphoreType.DMA] * (3 * NB),
  )
  def msg_kernel(hs_hbm, edata_hbm, out0_hbm, out1_hbm, eb0, eb1, eb2, eb3,
                 eb4, rw0, rw1, rw2, rw3, rw4, zbuf, acc,
                 el0, el1, el2, el3, el4, gl0, gl1, gl2, gl3, gl4,
                 sl0, sl1, sl2, sl3, sl4):
    ebufs = (eb0, eb1, eb2, eb3, eb4)
    rows = (rw0, rw1, rw2, rw3, rw4)
    esem = (el0, el1, el2, el3, el4)
    gsem = (gl0, gl1, gl2, gl3, gl4)
    ssem = (sl0, sl1, sl2, sl3, sl4)
    cid = lax.axis_index("c")
    sid = lax.axis_index("s")
    w = cid * NS + sid
    for i in range(zrows):
      zbuf[i] = jnp.zeros((16,), F32)

    @pl.loop(sid, nzc, step=NS)
    def _(j):
      pltpu.sync_copy(zbuf, acc.at[pl.ds(j * zrows, zrows)])

    plsc.subcore_barrier()
    base, n = _worker_span(w, nrows_total)

    for k in range(3):
      pltpu.async_copy(edata_hbm.at[base + k], ebufs[k], esem[k])
    for k in range(2):
      pltpu.make_async_copy(edata_hbm.at[base + k], ebufs[k], esem[k]).wait()
      pltpu.async_copy(hs_hbm.at[ebufs[k].at[0]], rows[k], gsem[k])

    @pl.loop(0, (n + NB - 1) // NB)
    def _(i):
      for b0 in range(NB):
        s = i * NB + b0
        b2 = (b0 + 2) % NB
        b3 = (b0 + 3) % NB

        @pl.when(s < n)
        def _():
          pltpu.make_async_copy(hs_hbm.at[ebufs[b0].at[0]], rows[b0],
                                gsem[b0]).wait()

        @pl.when(jnp.logical_and(s >= 2, s < n))
        def _():
          pltpu.make_async_copy(rows[b3], acc.at[ebufs[b3].at[1]],
                                ssem[b3]).wait()

        @pl.when(s + 3 < n)
        def _():
          pltpu.async_copy(edata_hbm.at[base + s + 3], ebufs[b3], esem[b3])

        @pl.when(s + 2 < n)
        def _():
          pltpu.make_async_copy(edata_hbm.at[base + s + 2], ebufs[b2],
                                esem[b2]).wait()
          pltpu.async_copy(hs_hbm.at[ebufs[b2].at[0]], rows[b2], gsem[b2])

        @pl.when(s < n)
        def _():
          ones_chunk = jnp.full((16,), one_f32_bits, I32)
          all_ones = jnp.all(ebufs[b0][2, pl.ds(0, 16)] == ones_chunk)
          for gch in range(1, EK // 16):
            all_ones = jnp.logical_and(
                all_ones,
                jnp.all(ebufs[b0][2, pl.ds(gch * 16, 16)] == ones_chunk))

          @pl.when(jnp.logical_not(all_ones))
          def _():
            for c in range(EK):
              mb = plsc.load_gather(
                  ebufs[b0],
                  [jnp.full((16,), 2, I32), jnp.full((16,), c, I32)])
              rows[b0][c] = rows[b0][c] * plsc.bitcast(mb, F32)

          pltpu.async_copy(rows[b0], acc.at[ebufs[b0].at[1]], ssem[b0],
                           add=True)

    for b in range(NB):
      for tail in (n - 2, n - 1):
        @pl.when(tail % NB == b)
        def _():
          pltpu.make_async_copy(rows[b], acc.at[ebufs[b].at[1]],
                                ssem[b]).wait()

    plsc.subcore_barrier()
    for cc, out_hbm in enumerate((out0_hbm, out1_hbm)):
      @pl.when(cid == cc)
      def _():
        @pl.loop(0, rps // zrows)
        def _(j):
          off = sid * rps + j * zrows
          pltpu.sync_copy(acc.at[pl.ds(off, zrows)], zbuf)
          pltpu.sync_copy(zbuf, out_hbm.at[pl.ds(off, zrows)])

  return msg_kernel(hs, edata)


def _tc_hs(deg0, deg1, x1, x2, rd, W1):
  n = x1.shape[0]
  g = n // BT

  def body(d0_r, d1_r, x1_r, x2_r, rd_r, w_r, hs_r):
    w = w_r[...]
    h = (jnp.dot(x1_r[...], w[0:16], preferred_element_type=F32)
         + jnp.dot(x2_r[...], w[16:32], preferred_element_type=F32)
         + jnp.dot(rd_r[...], w[32:48], preferred_element_type=F32))
    dinv = _dinv(d0_r[0, 0, :] + d1_r[0, 0, :])
    hs_r[...] = h * dinv[:, None]

  return pl.pallas_call(
      body,
      grid=(g,),
      in_specs=[pl.BlockSpec((1, 1, BT), lambda i: (i, 0, 0))] * 2
      + [pl.BlockSpec((BT, 16), lambda i: (i, 0))] * 3
      + [pl.BlockSpec((48, 16), lambda i: (0, 0))],
      out_specs=pl.BlockSpec((BT, D), lambda i: (i, 0)),
      out_shape=jax.ShapeDtypeStruct((n, D), F32),
  )(deg0, deg1, x1, x2, rd, W1)


def _dinv(d):
  return jnp.where(d > 0, 1.0 / jnp.sqrt(jnp.maximum(d, 1e-12)), 0.0)


def _elu(x):
  return jnp.where(x > 0, x, jnp.exp(jnp.minimum(x, 0.0)) - 1.0)


def _stats_update(st_r, y):
  s = jnp.sum(y, axis=0)
  ss = jnp.sum(y * y, axis=0)
  upd = jnp.concatenate(
      [s[None, :], ss[None, :], jnp.zeros((6, y.shape[1]), F32)], axis=0)
  st_r[...] += upd


def _fold(st_r, g, b, nf):
  mu = st_r[0, :] / nf
  var = jnp.maximum(st_r[1, :] / nf - mu * mu, 0.0)
  si = g / jnp.sqrt(var + 1e-5)
  return si, b - mu * si


def _tc_conv_finish(deg0, deg1, p0, p1, x1, x2, b1):
  n = x1.shape[0]
  g = n // BT

  def body(d0_r, d1_r, p0_r, p1_r, x1_r, x2_r, b1_r, he_r, st_r):
    @pl.when(pl.program_id(0) == 0)
    def _():
      st_r[...] = jnp.zeros_like(st_r)

    dinv = _dinv(d0_r[0, 0, :] + d1_r[0, 0, :])
    conv = (p0_r[...] + p1_r[...]) * dinv[:, None] + b1_r[...]
    he = _elu(conv)
    he_r[...] = he
    y = jnp.concatenate([x1_r[...], x2_r[...], he], axis=1)
    _stats_update(st_r, y)

  return pl.pallas_call(
      body,
      grid=(g,),
      in_specs=[pl.BlockSpec((1, 1, BT), lambda i: (i, 0, 0))] * 2
      + [pl.BlockSpec((BT, D), lambda i: (i, 0))] * 4
      + [pl.BlockSpec((1, D), lambda i: (0, 0))],
      out_specs=(pl.BlockSpec((BT, D), lambda i: (i, 0)),
                 pl.BlockSpec((8, 48), lambda i: (0, 0))),
      out_shape=(jax.ShapeDtypeStruct((n, D), F32),
                 jax.ShapeDtypeStruct((8, 48), F32)),
  )(deg0, deg1, p0, p1, x1, x2, b1)


def _tc_mlp0(x1, x2, he, A0, c0):
  n = x1.shape[0]
  g = n // BT

  def body(x1_r, x2_r, he_r, a_r, c_r, a0_r, st_r):
    @pl.when(pl.program_id(0) == 0)
    def _():
      st_r[...] = jnp.zeros_like(st_r)

    y = jnp.concatenate([x1_r[...], x2_r[...], he_r[...]], axis=1)
    a0 = _elu(jnp.dot(y, a_r[...], preferred_element_type=F32) + c_r[...])
    a0_r[...] = a0
    _stats_update(st_r, a0)

  return pl.pallas_call(
      body,
      grid=(g,),
      in_specs=[pl.BlockSpec((BT, D), lambda i: (i, 0))] * 3
      + [pl.BlockSpec((48, 16), lambda i: (0, 0)),
         pl.BlockSpec((1, 16), lambda i: (0, 0))],
      out_specs=(pl.BlockSpec((BT, 16), lambda i: (i, 0)),
                 pl.BlockSpec((8, 16), lambda i: (0, 0))),
      out_shape=(jax.ShapeDtypeStruct((n, 16), F32),
                 jax.ShapeDtypeStruct((8, 16), F32)),
  )(x1, x2, he, A0, c0)


def _tc_mlp1(a0, A1, c1):
  n = a0.shape[0]
  g = n // BT

  def body(a0_r, a_r, c_r, a1_r, st_r):
    @pl.when(pl.program_id(0) == 0)
    def _():
      st_r[...] = jnp.zeros_like(st_r)

    a1 = _elu(jnp.dot(a0_r[...], a_r[...], preferred_element_type=F32)
              + c_r[...])
    a1_r[...] = a1
    _stats_update(st_r, a1)

  return pl.pallas_call(
      body,
      grid=(g,),
      in_specs=[pl.BlockSpec((BT, 16), lambda i: (i, 0)),
                pl.BlockSpec((16, 8), lambda i: (0, 0)),
                pl.BlockSpec((1, 8), lambda i: (0, 0))],
      out_specs=(pl.BlockSpec((BT, 8), lambda i: (i, 0)),
                 pl.BlockSpec((8, 8), lambda i: (0, 0))),
      out_shape=(jax.ShapeDtypeStruct((n, 8), F32),
                 jax.ShapeDtypeStruct((8, 8), F32)),
  )(a0, A1, c1)


def _tc_affine(a1, scale, off):
  n = a1.shape[0]
  g = n // BT

  def body(a1_r, s_r, o_r, out_r):
    out_r[...] = a1_r[...] * s_r[...] + o_r[...]

  return pl.pallas_call(
      body,
      grid=(g,),
      in_specs=[pl.BlockSpec((BT, 8), lambda i: (i, 0)),
                pl.BlockSpec((1, 8), lambda i: (0, 0)),
                pl.BlockSpec((1, 8), lambda i: (0, 0))],
      out_specs=pl.BlockSpec((BT, 8), lambda i: (i, 0)),
      out_shape=jax.ShapeDtypeStruct((n, 8), F32),
  )(a1, scale, off)


def kernel(x1, x2, batch, random_dims, x_j_mask, edge_index, W1, b1,
           bn1_g, bn1_b, mlp0_W, mlp0_b, bnm0_g, bnm0_b,
           mlp1_W, mlp1_b, bnm1_g, bnm1_b):
  del batch
  n = x1.shape[0]
  e = x_j_mask.shape[0]
  r = e // EK
  src = edge_index[0].astype(I32)
  dst = edge_index[1].astype(I32)
  dst_rows = dst.reshape(r, EK)
  mbits = lax.bitcast_convert_type(x_j_mask.astype(F32), I32)
  edata = jnp.stack([src.reshape(r, EK), dst_rows, mbits.reshape(r, EK)],
                    axis=1)

  deg0f, deg1f = _deg_call(dst_rows.reshape(r, 1, EK), n)
  d0 = deg0f.reshape(n // BT, 1, BT)
  d1 = deg1f.reshape(n // BT, 1, BT)
  hs = _tc_hs(d0, d1, x1, x2, random_dims, W1)
  p0, p1 = _msg_call(hs, edata, n)

  he, yst = _tc_conv_finish(d0, d1, p0, p1, x1, x2, b1.reshape(1, D))

  nf = float(n)
  mu0 = yst[0] / nf
  var0 = jnp.maximum(yst[1] / nf - mu0 * mu0, 0.0)
  s0inv = bn1_g / jnp.sqrt(var0 + 1e-5)
  A0 = s0inv[:, None] * mlp0_W
  c0 = (bn1_b - mu0 * s0inv) @ mlp0_W + mlp0_b

  a0, st0 = _tc_mlp0(x1, x2, he, A0, c0.reshape(1, 16))
  mu1 = st0[0] / nf
  var1 = jnp.maximum(st0[1] / nf - mu1 * mu1, 0.0)
  s1inv = bnm0_g / jnp.sqrt(var1 + 1e-5)
  A1 = s1inv[:, None] * mlp1_W
  c1 = (bnm0_b - mu1 * s1inv) @ mlp1_W + mlp1_b

  a1, st1 = _tc_mlp1(a0, A1, c1.reshape(1, 8))
  mu2 = st1[0] / nf
  var2 = jnp.maximum(st1[1] / nf - mu2 * mu2, 0.0)
  s2inv = bnm1_g / jnp.sqrt(var2 + 1e-5)
  off2 = bnm1_b - mu2 * s2inv

  return _tc_affine(a1, s2inv.reshape(1, 8), off2.reshape(1, 8))
